# baseline (speedup 1.0000x reference)
import jax
import jax.numpy as jnp
from jax import lax
from jax.experimental import pallas as pl
from jax.experimental.pallas import tpu as pltpu
from jax.experimental.pallas import tpu_sc as plsc

MODEL_DIM = 64
SCALE = 8.0

NC = 2
NS = 16
NW = NC * NS
LANE = 16
IDX_W = 128
KG = 4
C = KG * IDX_W


def _body(idx_hbm, table_hbm, out_hbm, idx_v, rows_v, gsem, n_groups):
    wid = lax.axis_index("s") * NC + lax.axis_index("c")
    row_base = wid * (n_groups * KG)
    out_base = wid * (n_groups * C)

    @pl.loop(0, n_groups)
    def _group(g):
        pltpu.sync_copy(idx_hbm.at[pl.ds(row_base + g * KG, KG)], idx_v)
        descs = []
        for j in range(KG):
            descs.append(
                pltpu.async_copy(
                    table_hbm.at[idx_v.at[j]],
                    rows_v.at[pl.ds(j * IDX_W, IDX_W)],
                    gsem,
                )
            )
        for d in descs:
            d.wait()

        @pl.loop(0, C)
        def _row(r):
            for c in range(MODEL_DIM // LANE):
                sl = pl.ds(c * LANE, LANE)
                rows_v[r, sl] = rows_v[r, sl] * SCALE

        pltpu.sync_copy(rows_v, out_hbm.at[pl.ds(out_base + g * C, C)])


def kernel(x, table):
    S0, S1 = x.shape
    B = S0 * S1
    n_groups = B // (NW * C)
    assert B == NW * C * n_groups

    idx = x.reshape(B // IDX_W, IDX_W).astype(jnp.int32)

    import functools

    run = pl.kernel(
        functools.partial(_body, n_groups=n_groups),
        out_type=jax.ShapeDtypeStruct((B, MODEL_DIM), jnp.float32),
        mesh=plsc.VectorSubcoreMesh(core_axis_name="c", subcore_axis_name="s"),
        scratch_types=[
            pltpu.VMEM((KG, IDX_W), jnp.int32),
            pltpu.VMEM((C, MODEL_DIM), jnp.float32),
            pltpu.SemaphoreType.DMA,
        ],
        compiler_params=pltpu.CompilerParams(use_tc_tiling_on_sc=False),
    )
    out = run(idx, table)
    return out.reshape(S0, S1, MODEL_DIM)
